# baseline (speedup 1.0000x reference)
import functools

import jax
import jax.numpy as jnp
from jax import lax
from jax.experimental import pallas as pl
from jax.experimental.pallas import tpu as pltpu
from jax.experimental.pallas import tpu_sc as plsc

N = 10000
E = 160000
D_IN = 300
HEADS = 4
D_HEAD = 16
D_MID = HEADS * D_HEAD
D_OUT = 32

NC = 2
NS = 16
NW = NC * NS

NPAD = 10240
BLK = 128
NBT = E // BLK
NT = 40
RPT = NPAD // NS

ROW_BLK = 1280

_SC_PARAMS = pltpu.CompilerParams(needs_layout_passes=False,
                                  use_tc_tiling_on_sc=False)


def _mm_a_kernel(x_ref, w_ref, a_ref, h_ref, al_ref):
    i = pl.program_id(0)
    rows = i * ROW_BLK + lax.broadcasted_iota(jnp.int32, (ROW_BLK, D_IN), 0)
    xm = jnp.where(rows < N, x_ref[...], 0.0)
    h = jnp.dot(xm, w_ref[...], preferred_element_type=jnp.float32)
    h_ref[...] = h
    al_ref[...] = jnp.dot(h, a_ref[...], preferred_element_type=jnp.float32)


def _tc_a(x, W1, A8):
    return pl.pallas_call(
        _mm_a_kernel,
        grid=(NPAD // ROW_BLK,),
        in_specs=[
            pl.BlockSpec((ROW_BLK, D_IN), lambda i: (i, 0)),
            pl.BlockSpec((D_IN, D_MID), lambda i: (0, 0)),
            pl.BlockSpec((D_MID, 8), lambda i: (0, 0)),
        ],
        out_specs=[
            pl.BlockSpec((ROW_BLK, D_MID), lambda i: (i, 0)),
            pl.BlockSpec((ROW_BLK, 8), lambda i: (i, 0)),
        ],
        out_shape=[
            jax.ShapeDtypeStruct((NPAD, D_MID), jnp.float32),
            jax.ShapeDtypeStruct((NPAD, 8), jnp.float32),
        ],
    )(x, W1, A8)


def _tile_range(wid):
    start = (wid * (NBT // 2)) >> 4
    nxt = ((wid + 1) * (NBT // 2)) >> 4
    return start, nxt - start


def _sc1_make():
    mesh = plsc.VectorSubcoreMesh(core_axis_name="c", subcore_axis_name="s",
                                  num_cores=NC, num_subcores=NS)

    @functools.partial(
        pl.kernel,
        out_type=[
            jax.ShapeDtypeStruct((NPAD, 2 * D_MID), jnp.float32),
            jax.ShapeDtypeStruct((NPAD, 32), jnp.float32),
        ],
        mesh=mesh,
        scratch_types=[
            pltpu.VMEM((NT, BLK), jnp.int32),
            pltpu.VMEM((NT, BLK), jnp.int32),
            pltpu.VMEM((NT, BLK), jnp.int32),
            pltpu.VMEM((4, BLK, D_MID), jnp.float32),
            pltpu.VMEM((4, BLK, 8), jnp.float32),
            pltpu.VMEM((4, BLK, 8), jnp.float32),
            pltpu.VMEM((4, BLK, 8), jnp.float32),
            pltpu.VMEM((BLK, 8), jnp.float32),
            pltpu.VMEM_SHARED((NPAD, D_MID), jnp.float32),
            pltpu.VMEM_SHARED((NPAD, 8), jnp.float32),
            pltpu.VMEM_SHARED((NPAD, 8), jnp.float32),
            pltpu.SemaphoreType.DMA,
            pltpu.SemaphoreType.DMA,
        ],
        compiler_params=_SC_PARAMS,
    )
    def sc1(h_hbm, al8_hbm, ei1_hbm, ei2_hbm,
            numer_out, small_out,
            src_v, dst_v, dst2_v, hs4, wpad4, asb4, adb4, ones8,
            numer_sh, wsum_sh, deg_sh, gsem, ssem):
        c = lax.axis_index("c")
        s = lax.axis_index("s")
        wid = c * NS + s
        start, nt = _tile_range(wid)
        il = lax.iota(jnp.int32, 16)
        zeros16 = jnp.zeros((16,), jnp.float32)
        ones_row = jnp.where((il & 7) == 0, 1.0, 0.0)

        hs = [hs4.at[k] for k in range(4)]
        wpad = [wpad4.at[k] for k in range(4)]
        asb = [asb4.at[k] for k in range(4)]
        adb = [adb4.at[k] for k in range(4)]

        def _sfire(t, _):
            pltpu.async_copy(ei1_hbm.at[0, pl.ds((start + t) * BLK, BLK)],
                             src_v.at[t], gsem)
            pltpu.async_copy(ei1_hbm.at[1, pl.ds((start + t) * BLK, BLK)],
                             dst_v.at[t], gsem)
            pltpu.async_copy(ei2_hbm.at[1, pl.ds((start + t) * BLK, BLK)],
                             dst2_v.at[t], gsem)
            return 0
        lax.fori_loop(0, NT, _sfire, 0)

        def _sdrain(t, _):
            pltpu.make_async_copy(ei1_hbm.at[0, pl.ds((start + t) * BLK, BLK)],
                                  src_v.at[t], gsem).wait()
            pltpu.make_async_copy(ei1_hbm.at[1, pl.ds((start + t) * BLK, BLK)],
                                  dst_v.at[t], gsem).wait()
            pltpu.make_async_copy(ei2_hbm.at[1, pl.ds((start + t) * BLK, BLK)],
                                  dst2_v.at[t], gsem).wait()
            return 0
        lax.fori_loop(0, NT, _sdrain, 0)

        def _zw(k, _):
            r8 = (k * 16 + il) >> 3
            c8 = il & 7
            for b in range(4):
                plsc.store_scatter(wpad[b], [r8, c8], zeros16)
            plsc.store_scatter(ones8, [r8, c8], ones_row)
            return 0
        lax.fori_loop(0, BLK * 8 // 16, _zw, 0)

        def _zh(i, _):
            for hh in range(HEADS):
                hs[0][i, pl.ds(hh * 16, 16)] = zeros16
            return 0
        lax.fori_loop(0, BLK, _zh, 0)

        t0 = s * RPT
        for r0 in range(0, RPT, BLK):
            pltpu.sync_copy(hs[0], numer_sh.at[pl.ds(t0 + r0, BLK)])
            pltpu.sync_copy(wpad[0], wsum_sh.at[pl.ds(t0 + r0, BLK)])
            pltpu.sync_copy(wpad[0], deg_sh.at[pl.ds(t0 + r0, BLK)])
        plsc.subcore_barrier()

        def _fire_gathers(t, b):
            pltpu.async_copy(h_hbm.at[src_v.at[t]], hs[b], gsem)
            pltpu.async_copy(al8_hbm.at[src_v.at[t]], asb[b], gsem)
            pltpu.async_copy(al8_hbm.at[dst_v.at[t]], adb[b], gsem)

        def _drain_gathers(t, b):
            pltpu.make_async_copy(h_hbm.at[src_v.at[t]], hs[b], gsem).wait()
            pltpu.make_async_copy(al8_hbm.at[src_v.at[t]], asb[b], gsem).wait()
            pltpu.make_async_copy(al8_hbm.at[dst_v.at[t]], adb[b], gsem).wait()

        def _fire_scatters(t, b):
            pltpu.async_copy(hs[b], numer_sh.at[dst_v.at[t]], ssem, add=True)
            pltpu.async_copy(wpad[b], wsum_sh.at[dst_v.at[t]], ssem, add=True)
            pltpu.async_copy(ones8, deg_sh.at[dst2_v.at[t]], ssem, add=True)

        def _drain_scatters(t, b):
            pltpu.make_async_copy(hs[b], numer_sh.at[dst_v.at[t]], ssem).wait()
            pltpu.make_async_copy(wpad[b], wsum_sh.at[dst_v.at[t]], ssem).wait()
            pltpu.make_async_copy(ones8, deg_sh.at[dst2_v.at[t]], ssem).wait()

        def _compute(t, b):
            def _wvec(k, _):
                base = k * 16
                e_l = (base + il) >> 2
                h_l = (base + il) & 3
                a = (plsc.load_gather(asb[b], [e_l, h_l])
                     + plsc.load_gather(adb[b], [e_l, h_l + 4]))
                a = jnp.where(a > 0, a, 0.2 * a)
                w = jnp.exp(a)
                plsc.store_scatter(wpad[b], [e_l, h_l], w)
                e0 = 4 * k
                for cc in range(4):
                    for hh in range(HEADS):
                        wv = w.at[jnp.full((16,), 4 * cc + hh, jnp.int32)].get(
                            mode="promise_in_bounds")
                        hs[b][e0 + cc, pl.ds(hh * 16, 16)] = (
                            hs[b][e0 + cc, pl.ds(hh * 16, 16)] * wv)
                return 0
            lax.fori_loop(0, BLK * HEADS // 16, _wvec, 0)

        @pl.when(nt > 0)
        def _():
            _fire_gathers(0, 0)

        @pl.when(nt > 1)
        def _():
            _fire_gathers(1, 1)

        def _step(t, b):
            @pl.when(jnp.logical_and(t >= 2, t - 2 < nt))
            def _():
                _drain_scatters(t - 2, (b + 2) % 4)

            @pl.when(t + 2 < nt)
            def _():
                _fire_gathers(t + 2, (b + 2) % 4)

            @pl.when(t < nt)
            def _():
                _drain_gathers(t, b)
                _compute(t, b)
                _fire_scatters(t, b)

        def _quad(q, _):
            for k in range(4):
                _step(4 * q + k, k)
            return 0
        lax.fori_loop(0, (NT + 2 + 3) // 4, _quad, 0)

        plsc.subcore_barrier()
        pltpu.sync_copy(numer_sh.at[pl.ds(t0, RPT)],
                        numer_out.at[pl.ds(t0, RPT), pl.ds(D_MID * c, D_MID)])
        pltpu.sync_copy(wsum_sh.at[pl.ds(t0, RPT)],
                        small_out.at[pl.ds(t0, RPT), pl.ds(8 * c, 8)])
        pltpu.sync_copy(deg_sh.at[pl.ds(t0, RPT)],
                        small_out.at[pl.ds(t0, RPT), pl.ds(16 + 8 * c, 8)])

    return sc1


def _tc_b_kernel(h_ref, al_ref, np_ref, sm_ref, r_ref, b1_ref, w2_ref,
                 h2p_ref):
    al = al_ref[...]
    sm = sm_ref[...]
    npc = np_ref[...]
    ws = jnp.exp(jax.nn.leaky_relu(al[:, :HEADS] + al[:, HEADS:], 0.2))
    wbc = jnp.dot(ws, r_ref[...], preferred_element_type=jnp.float32)
    numer = npc[:, :D_MID] + npc[:, D_MID:] + wbc * h_ref[...]
    den = sm[:, :HEADS] + sm[:, 8:8 + HEADS] + ws
    den_bc = jnp.dot(den, r_ref[...], preferred_element_type=jnp.float32)
    out1 = jax.nn.relu(numer / (den_bc + 1e-16) + b1_ref[...])
    h2 = jnp.dot(out1, w2_ref[...], preferred_element_type=jnp.float32)
    deg = sm[:, 16:17] + sm[:, 24:25] + 1.0
    h2p_ref[...] = lax.rsqrt(deg) * h2


def _tc_b(h, al8, numer_c, small_c, R, b1r, W2):
    return pl.pallas_call(
        _tc_b_kernel,
        grid=(NPAD // 640,),
        in_specs=[
            pl.BlockSpec((640, D_MID), lambda i: (i, 0)),
            pl.BlockSpec((640, 8), lambda i: (i, 0)),
            pl.BlockSpec((640, 2 * D_MID), lambda i: (i, 0)),
            pl.BlockSpec((640, 32), lambda i: (i, 0)),
            pl.BlockSpec((HEADS, D_MID), lambda i: (0, 0)),
            pl.BlockSpec((1, D_MID), lambda i: (0, 0)),
            pl.BlockSpec((D_MID, D_OUT), lambda i: (0, 0)),
        ],
        out_specs=pl.BlockSpec((640, D_OUT), lambda i: (i, 0)),
        out_shape=jax.ShapeDtypeStruct((NPAD, D_OUT), jnp.float32),
    )(h, al8, numer_c, small_c, R, b1r, W2)


def _sc2_make():
    mesh = plsc.VectorSubcoreMesh(core_axis_name="c", subcore_axis_name="s",
                                  num_cores=NC, num_subcores=NS)

    @functools.partial(
        pl.kernel,
        out_type=jax.ShapeDtypeStruct((NC, NPAD, D_OUT), jnp.float32),
        mesh=mesh,
        scratch_types=[
            pltpu.VMEM((NT, BLK), jnp.int32),
            pltpu.VMEM((NT, BLK), jnp.int32),
            pltpu.VMEM((4, BLK, D_OUT), jnp.float32),
            pltpu.VMEM_SHARED((NPAD, D_OUT), jnp.float32),
            pltpu.SemaphoreType.DMA,
            pltpu.SemaphoreType.DMA,
        ],
        compiler_params=_SC_PARAMS,
    )
    def sc2(h2p_hbm, ei2_hbm, out, src_v, dst_v, buf4, acc_sh, gsem, ssem):
        c = lax.axis_index("c")
        s = lax.axis_index("s")
        wid = c * NS + s
        start, nt = _tile_range(wid)
        zeros16 = jnp.zeros((16,), jnp.float32)
        buf = [buf4.at[k] for k in range(4)]

        def _sfire(t, _):
            pltpu.async_copy(ei2_hbm.at[0, pl.ds((start + t) * BLK, BLK)],
                             src_v.at[t], gsem)
            pltpu.async_copy(ei2_hbm.at[1, pl.ds((start + t) * BLK, BLK)],
                             dst_v.at[t], gsem)
            return 0
        lax.fori_loop(0, NT, _sfire, 0)

        def _sdrain(t, _):
            pltpu.make_async_copy(ei2_hbm.at[0, pl.ds((start + t) * BLK, BLK)],
                                  src_v.at[t], gsem).wait()
            pltpu.make_async_copy(ei2_hbm.at[1, pl.ds((start + t) * BLK, BLK)],
                                  dst_v.at[t], gsem).wait()
            return 0
        lax.fori_loop(0, NT, _sdrain, 0)

        def _zrow(i, _):
            for hh in range(D_OUT // 16):
                buf[0][i, pl.ds(hh * 16, 16)] = zeros16
            return 0
        lax.fori_loop(0, BLK, _zrow, 0)

        t0 = s * RPT
        for r0 in range(0, RPT, BLK):
            pltpu.sync_copy(buf[0], acc_sh.at[pl.ds(t0 + r0, BLK)])
        plsc.subcore_barrier()

        def _fire_g(t, b):
            pltpu.async_copy(h2p_hbm.at[src_v.at[t]], buf[b], gsem)

        def _drain_g(t, b):
            pltpu.make_async_copy(h2p_hbm.at[src_v.at[t]], buf[b], gsem).wait()

        def _fire_s(t, b):
            pltpu.async_copy(buf[b], acc_sh.at[dst_v.at[t]], ssem, add=True)

        def _drain_s(t, b):
            pltpu.make_async_copy(buf[b], acc_sh.at[dst_v.at[t]], ssem).wait()

        @pl.when(nt > 0)
        def _():
            _fire_g(0, 0)

        @pl.when(nt > 1)
        def _():
            _fire_g(1, 1)

        def _step(t, b):
            @pl.when(jnp.logical_and(t >= 2, t - 2 < nt))
            def _():
                _drain_s(t - 2, (b + 2) % 4)

            @pl.when(t + 2 < nt)
            def _():
                _fire_g(t + 2, (b + 2) % 4)

            @pl.when(t < nt)
            def _():
                _drain_g(t, b)
                _fire_s(t, b)

        def _quad(q, _):
            for k in range(4):
                _step(4 * q + k, k)
            return 0
        lax.fori_loop(0, (NT + 2 + 3) // 4, _quad, 0)

        plsc.subcore_barrier()
        pltpu.sync_copy(acc_sh.at[pl.ds(t0, RPT)], out.at[c, pl.ds(t0, RPT)])

    return sc2


ROW_BLK_C = 1000


def _tc_c_kernel(p_ref, h2p_ref, sm_ref, b2_ref, out_ref):
    sm = sm_ref[...]
    deg = sm[:, 16:17] + sm[:, 24:25] + 1.0
    out_ref[...] = lax.rsqrt(deg) * (p_ref[0] + p_ref[1] + h2p_ref[...]) + b2_ref[...]


def _tc_c(out2_p, h2p, small_c, b2r):
    return pl.pallas_call(
        _tc_c_kernel,
        grid=(N // ROW_BLK_C,),
        in_specs=[
            pl.BlockSpec((NC, ROW_BLK_C, D_OUT), lambda i: (0, i, 0)),
            pl.BlockSpec((ROW_BLK_C, D_OUT), lambda i: (i, 0)),
            pl.BlockSpec((ROW_BLK_C, 32), lambda i: (i, 0)),
            pl.BlockSpec((1, D_OUT), lambda i: (0, 0)),
        ],
        out_specs=pl.BlockSpec((ROW_BLK_C, D_OUT), lambda i: (i, 0)),
        out_shape=jax.ShapeDtypeStruct((N, D_OUT), jnp.float32),
    )(out2_p, h2p, small_c, b2r)


def kernel(x, edge_index, sec_order_edge_index, W1, att_src, att_dst, b1, W2, b2):
    f32 = jnp.float32
    Asrc = (jnp.eye(HEADS, dtype=f32)[:, None, :] * att_src[:, :, None]).reshape(D_MID, HEADS)
    Adst = (jnp.eye(HEADS, dtype=f32)[:, None, :] * att_dst[:, :, None]).reshape(D_MID, HEADS)
    A8 = jnp.concatenate([Asrc, Adst], axis=1)
    R = jnp.repeat(jnp.eye(HEADS, dtype=f32), D_HEAD, axis=1)

    ei1 = edge_index
    ei2 = sec_order_edge_index

    h, al8 = _tc_a(x, W1, A8)
    numer_c, small_c = _sc1_make()(h, al8, ei1, ei2)
    h2p = _tc_b(h, al8, numer_c, small_c, R, b1.reshape(1, D_MID), W2)
    out2_p = _sc2_make()(h2p, ei2)
    return _tc_c(out2_p, h2p, small_c, b2.reshape(1, D_OUT))


# speedup vs baseline: 1.0208x; 1.0208x over previous
import functools

import jax
import jax.numpy as jnp
from jax import lax
from jax.experimental import pallas as pl
from jax.experimental.pallas import tpu as pltpu
from jax.experimental.pallas import tpu_sc as plsc

N = 10000
E = 160000
D_IN = 300
HEADS = 4
D_HEAD = 16
D_MID = HEADS * D_HEAD
D_OUT = 32

NC = 2
NS = 16
NW = NC * NS

NPAD = 10240
BLK = 128
NBT = E // BLK
NT = 40
RPT = NPAD // NS

ROW_BLK = 1280

_SC_PARAMS = pltpu.CompilerParams(needs_layout_passes=False,
                                  use_tc_tiling_on_sc=False)


def _mm_a_kernel(x_ref, w_ref, a_ref, h_ref, al_ref):
    i = pl.program_id(0)
    rows = i * ROW_BLK + lax.broadcasted_iota(jnp.int32, (ROW_BLK, D_IN), 0)
    xm = jnp.where(rows < N, x_ref[...], 0.0)
    h = jnp.dot(xm, w_ref[...], preferred_element_type=jnp.float32)
    h_ref[...] = h
    al_ref[...] = jnp.dot(h, a_ref[...], preferred_element_type=jnp.float32)


def _tc_a(x, W1, A8):
    return pl.pallas_call(
        _mm_a_kernel,
        grid=(NPAD // ROW_BLK,),
        in_specs=[
            pl.BlockSpec((ROW_BLK, D_IN), lambda i: (i, 0)),
            pl.BlockSpec((D_IN, D_MID), lambda i: (0, 0)),
            pl.BlockSpec((D_MID, 8), lambda i: (0, 0)),
        ],
        out_specs=[
            pl.BlockSpec((ROW_BLK, D_MID), lambda i: (i, 0)),
            pl.BlockSpec((ROW_BLK, 8), lambda i: (i, 0)),
        ],
        out_shape=[
            jax.ShapeDtypeStruct((NPAD, D_MID), jnp.float32),
            jax.ShapeDtypeStruct((NPAD, 8), jnp.float32),
        ],
    )(x, W1, A8)


def _tile_range(wid):
    start = (wid * (NBT // 2)) >> 4
    nxt = ((wid + 1) * (NBT // 2)) >> 4
    return start, nxt - start


def _sc1_make():
    mesh = plsc.VectorSubcoreMesh(core_axis_name="c", subcore_axis_name="s",
                                  num_cores=NC, num_subcores=NS)

    @functools.partial(
        pl.kernel,
        out_type=[
            jax.ShapeDtypeStruct((NPAD, 2 * D_MID), jnp.float32),
            jax.ShapeDtypeStruct((NPAD, 32), jnp.float32),
        ],
        mesh=mesh,
        scratch_types=[
            pltpu.VMEM((NT, BLK), jnp.int32),
            pltpu.VMEM((NT, BLK), jnp.int32),
            pltpu.VMEM((NT, BLK), jnp.int32),
            pltpu.VMEM((4, BLK, D_MID), jnp.float32),
            pltpu.VMEM((4, BLK, 8), jnp.float32),
            pltpu.VMEM((4, BLK, 8), jnp.float32),
            pltpu.VMEM((4, BLK, 8), jnp.float32),
            pltpu.VMEM((BLK, 8), jnp.float32),
            pltpu.VMEM_SHARED((NPAD, D_MID), jnp.float32),
            pltpu.VMEM_SHARED((NPAD, 8), jnp.float32),
            pltpu.VMEM_SHARED((NPAD, 8), jnp.float32),
            pltpu.SemaphoreType.DMA,
            pltpu.SemaphoreType.DMA,
        ],
        compiler_params=_SC_PARAMS,
    )
    def sc1(h_hbm, al8_hbm, ei1_hbm, ei2_hbm,
            numer_out, small_out,
            src_v, dst_v, dst2_v, hs4, wpad4, asb4, adb4, ones8,
            numer_sh, wsum_sh, deg_sh, gsem, ssem):
        c = lax.axis_index("c")
        s = lax.axis_index("s")
        wid = c * NS + s
        start, nt = _tile_range(wid)
        il = lax.iota(jnp.int32, 16)
        zeros16 = jnp.zeros((16,), jnp.float32)
        ones_row = jnp.where((il & 7) == 0, 1.0, 0.0)

        hs = [hs4.at[k] for k in range(4)]
        wpad = [wpad4.at[k] for k in range(4)]
        asb = [asb4.at[k] for k in range(4)]
        adb = [adb4.at[k] for k in range(4)]

        def _sfire(t, _):
            pltpu.async_copy(ei1_hbm.at[0, pl.ds((start + t) * BLK, BLK)],
                             src_v.at[t], gsem)
            pltpu.async_copy(ei1_hbm.at[1, pl.ds((start + t) * BLK, BLK)],
                             dst_v.at[t], gsem)
            pltpu.async_copy(ei2_hbm.at[1, pl.ds((start + t) * BLK, BLK)],
                             dst2_v.at[t], gsem)
            return 0
        lax.fori_loop(0, NT, _sfire, 0)

        def _sdrain(t, _):
            pltpu.make_async_copy(ei1_hbm.at[0, pl.ds((start + t) * BLK, BLK)],
                                  src_v.at[t], gsem).wait()
            pltpu.make_async_copy(ei1_hbm.at[1, pl.ds((start + t) * BLK, BLK)],
                                  dst_v.at[t], gsem).wait()
            pltpu.make_async_copy(ei2_hbm.at[1, pl.ds((start + t) * BLK, BLK)],
                                  dst2_v.at[t], gsem).wait()
            return 0
        lax.fori_loop(0, NT, _sdrain, 0)

        def _zw(k, _):
            r8 = (k * 16 + il) >> 3
            c8 = il & 7
            for b in range(4):
                plsc.store_scatter(wpad[b], [r8, c8], zeros16)
            plsc.store_scatter(ones8, [r8, c8], ones_row)
            return 0
        lax.fori_loop(0, BLK * 8 // 16, _zw, 0)

        def _zh(i, _):
            for hh in range(HEADS):
                hs[0][i, pl.ds(hh * 16, 16)] = zeros16
            return 0
        lax.fori_loop(0, BLK, _zh, 0)

        t0 = s * RPT
        for r0 in range(0, RPT, BLK):
            pltpu.sync_copy(hs[0], numer_sh.at[pl.ds(t0 + r0, BLK)])
            pltpu.sync_copy(wpad[0], wsum_sh.at[pl.ds(t0 + r0, BLK)])
            pltpu.sync_copy(wpad[0], deg_sh.at[pl.ds(t0 + r0, BLK)])
        plsc.subcore_barrier()

        def _fire_gathers(t, b):
            pltpu.async_copy(h_hbm.at[src_v.at[t]], hs[b], gsem)
            pltpu.async_copy(al8_hbm.at[src_v.at[t]], asb[b], gsem)
            pltpu.async_copy(al8_hbm.at[dst_v.at[t]], adb[b], gsem)

        def _drain_gathers(t, b):
            pltpu.make_async_copy(h_hbm.at[src_v.at[t]], hs[b], gsem).wait()
            pltpu.make_async_copy(al8_hbm.at[src_v.at[t]], asb[b], gsem).wait()
            pltpu.make_async_copy(al8_hbm.at[dst_v.at[t]], adb[b], gsem).wait()

        def _fire_scatters(t, b):
            pltpu.async_copy(hs[b], numer_sh.at[dst_v.at[t]], ssem, add=True)
            pltpu.async_copy(wpad[b], wsum_sh.at[dst_v.at[t]], ssem, add=True)
            pltpu.async_copy(ones8, deg_sh.at[dst2_v.at[t]], ssem, add=True)

        def _drain_scatters(t, b):
            pltpu.make_async_copy(hs[b], numer_sh.at[dst_v.at[t]], ssem).wait()
            pltpu.make_async_copy(wpad[b], wsum_sh.at[dst_v.at[t]], ssem).wait()
            pltpu.make_async_copy(ones8, deg_sh.at[dst2_v.at[t]], ssem).wait()

        def _compute(t, b):
            def _wvec(k, _):
                base = k * 16
                e_l = (base + il) >> 2
                h_l = (base + il) & 3
                a = (plsc.load_gather(asb[b], [e_l, h_l])
                     + plsc.load_gather(adb[b], [e_l, h_l + 4]))
                a = jnp.where(a > 0, a, 0.2 * a)
                w = jnp.exp(a)
                plsc.store_scatter(wpad[b], [e_l, h_l], w)
                e0 = 4 * k
                for cc in range(4):
                    for hh in range(HEADS):
                        wv = w.at[jnp.full((16,), 4 * cc + hh, jnp.int32)].get(
                            mode="promise_in_bounds")
                        hs[b][e0 + cc, pl.ds(hh * 16, 16)] = (
                            hs[b][e0 + cc, pl.ds(hh * 16, 16)] * wv)
                return 0
            lax.fori_loop(0, BLK * HEADS // 16, _wvec, 0)

        @pl.when(nt > 0)
        def _():
            _fire_gathers(0, 0)

        @pl.when(nt > 1)
        def _():
            _fire_gathers(1, 1)

        def _step(t, b):
            @pl.when(jnp.logical_and(t >= 2, t - 2 < nt))
            def _():
                _drain_scatters(t - 2, (b + 2) % 4)

            @pl.when(t + 2 < nt)
            def _():
                _fire_gathers(t + 2, (b + 2) % 4)

            @pl.when(t < nt)
            def _():
                _drain_gathers(t, b)
                _compute(t, b)
                _fire_scatters(t, b)

        def _quad(q, _):
            for k in range(4):
                _step(4 * q + k, k)
            return 0
        lax.fori_loop(0, (NT + 2 + 3) // 4, _quad, 0)

        plsc.subcore_barrier()
        pltpu.sync_copy(numer_sh.at[pl.ds(t0, RPT)],
                        numer_out.at[pl.ds(t0, RPT), pl.ds(D_MID * c, D_MID)])
        pltpu.sync_copy(wsum_sh.at[pl.ds(t0, RPT)],
                        small_out.at[pl.ds(t0, RPT), pl.ds(8 * c, 8)])
        pltpu.sync_copy(deg_sh.at[pl.ds(t0, RPT)],
                        small_out.at[pl.ds(t0, RPT), pl.ds(16 + 8 * c, 8)])

    return sc1


def _tc_b_kernel(h_ref, al_ref, np_ref, sm_ref, r_ref, b1_ref, w2_ref,
                 h2p_ref):
    al = al_ref[...]
    sm = sm_ref[...]
    npc = np_ref[...]
    ws = jnp.exp(jax.nn.leaky_relu(al[:, :HEADS] + al[:, HEADS:], 0.2))
    wbc = jnp.dot(ws, r_ref[...], preferred_element_type=jnp.float32)
    numer = npc[:, :D_MID] + npc[:, D_MID:] + wbc * h_ref[...]
    den = sm[:, :HEADS] + sm[:, 8:8 + HEADS] + ws
    den_bc = jnp.dot(den, r_ref[...], preferred_element_type=jnp.float32)
    out1 = jax.nn.relu(numer / (den_bc + 1e-16) + b1_ref[...])
    h2 = jnp.dot(out1, w2_ref[...], preferred_element_type=jnp.float32)
    deg = sm[:, 16:17] + sm[:, 24:25] + 1.0
    h2p_ref[...] = lax.rsqrt(deg) * h2


def _tc_b(h, al8, numer_c, small_c, R, b1r, W2):
    return pl.pallas_call(
        _tc_b_kernel,
        grid=(NPAD // ROW_BLK,),
        in_specs=[
            pl.BlockSpec((ROW_BLK, D_MID), lambda i: (i, 0)),
            pl.BlockSpec((ROW_BLK, 8), lambda i: (i, 0)),
            pl.BlockSpec((ROW_BLK, 2 * D_MID), lambda i: (i, 0)),
            pl.BlockSpec((ROW_BLK, 32), lambda i: (i, 0)),
            pl.BlockSpec((HEADS, D_MID), lambda i: (0, 0)),
            pl.BlockSpec((1, D_MID), lambda i: (0, 0)),
            pl.BlockSpec((D_MID, D_OUT), lambda i: (0, 0)),
        ],
        out_specs=pl.BlockSpec((ROW_BLK, D_OUT), lambda i: (i, 0)),
        out_shape=jax.ShapeDtypeStruct((NPAD, D_OUT), jnp.float32),
    )(h, al8, numer_c, small_c, R, b1r, W2)


def _sc2_make():
    mesh = plsc.VectorSubcoreMesh(core_axis_name="c", subcore_axis_name="s",
                                  num_cores=NC, num_subcores=NS)

    @functools.partial(
        pl.kernel,
        out_type=jax.ShapeDtypeStruct((NC, NPAD, D_OUT), jnp.float32),
        mesh=mesh,
        scratch_types=[
            pltpu.VMEM((NT, BLK), jnp.int32),
            pltpu.VMEM((NT, BLK), jnp.int32),
            pltpu.VMEM((4, BLK, D_OUT), jnp.float32),
            pltpu.VMEM_SHARED((NPAD, D_OUT), jnp.float32),
            pltpu.SemaphoreType.DMA,
            pltpu.SemaphoreType.DMA,
        ],
        compiler_params=_SC_PARAMS,
    )
    def sc2(h2p_hbm, ei2_hbm, out, src_v, dst_v, buf4, acc_sh, gsem, ssem):
        c = lax.axis_index("c")
        s = lax.axis_index("s")
        wid = c * NS + s
        start, nt = _tile_range(wid)
        zeros16 = jnp.zeros((16,), jnp.float32)
        buf = [buf4.at[k] for k in range(4)]

        def _sfire(t, _):
            pltpu.async_copy(ei2_hbm.at[0, pl.ds((start + t) * BLK, BLK)],
                             src_v.at[t], gsem)
            pltpu.async_copy(ei2_hbm.at[1, pl.ds((start + t) * BLK, BLK)],
                             dst_v.at[t], gsem)
            return 0
        lax.fori_loop(0, NT, _sfire, 0)

        def _sdrain(t, _):
            pltpu.make_async_copy(ei2_hbm.at[0, pl.ds((start + t) * BLK, BLK)],
                                  src_v.at[t], gsem).wait()
            pltpu.make_async_copy(ei2_hbm.at[1, pl.ds((start + t) * BLK, BLK)],
                                  dst_v.at[t], gsem).wait()
            return 0
        lax.fori_loop(0, NT, _sdrain, 0)

        def _zrow(i, _):
            for hh in range(D_OUT // 16):
                buf[0][i, pl.ds(hh * 16, 16)] = zeros16
            return 0
        lax.fori_loop(0, BLK, _zrow, 0)

        t0 = s * RPT
        for r0 in range(0, RPT, BLK):
            pltpu.sync_copy(buf[0], acc_sh.at[pl.ds(t0 + r0, BLK)])
        plsc.subcore_barrier()

        def _fire_g(t, b):
            pltpu.async_copy(h2p_hbm.at[src_v.at[t]], buf[b], gsem)

        def _drain_g(t, b):
            pltpu.make_async_copy(h2p_hbm.at[src_v.at[t]], buf[b], gsem).wait()

        def _fire_s(t, b):
            pltpu.async_copy(buf[b], acc_sh.at[dst_v.at[t]], ssem, add=True)

        def _drain_s(t, b):
            pltpu.make_async_copy(buf[b], acc_sh.at[dst_v.at[t]], ssem).wait()

        @pl.when(nt > 0)
        def _():
            _fire_g(0, 0)

        @pl.when(nt > 1)
        def _():
            _fire_g(1, 1)

        def _step(t, b):
            @pl.when(jnp.logical_and(t >= 2, t - 2 < nt))
            def _():
                _drain_s(t - 2, (b + 2) % 4)

            @pl.when(t + 2 < nt)
            def _():
                _fire_g(t + 2, (b + 2) % 4)

            @pl.when(t < nt)
            def _():
                _drain_g(t, b)
                _fire_s(t, b)

        def _quad(q, _):
            for k in range(4):
                _step(4 * q + k, k)
            return 0
        lax.fori_loop(0, (NT + 2 + 3) // 4, _quad, 0)

        plsc.subcore_barrier()
        pltpu.sync_copy(acc_sh.at[pl.ds(t0, RPT)], out.at[c, pl.ds(t0, RPT)])

    return sc2


ROW_BLK_C = 1000


def _tc_c_kernel(p_ref, h2p_ref, sm_ref, b2_ref, out_ref):
    sm = sm_ref[...]
    deg = sm[:, 16:17] + sm[:, 24:25] + 1.0
    out_ref[...] = lax.rsqrt(deg) * (p_ref[0] + p_ref[1] + h2p_ref[...]) + b2_ref[...]


def _tc_c(out2_p, h2p, small_c, b2r):
    return pl.pallas_call(
        _tc_c_kernel,
        grid=(N // ROW_BLK_C,),
        in_specs=[
            pl.BlockSpec((NC, ROW_BLK_C, D_OUT), lambda i: (0, i, 0)),
            pl.BlockSpec((ROW_BLK_C, D_OUT), lambda i: (i, 0)),
            pl.BlockSpec((ROW_BLK_C, 32), lambda i: (i, 0)),
            pl.BlockSpec((1, D_OUT), lambda i: (0, 0)),
        ],
        out_specs=pl.BlockSpec((ROW_BLK_C, D_OUT), lambda i: (i, 0)),
        out_shape=jax.ShapeDtypeStruct((N, D_OUT), jnp.float32),
    )(out2_p, h2p, small_c, b2r)


def kernel(x, edge_index, sec_order_edge_index, W1, att_src, att_dst, b1, W2, b2):
    f32 = jnp.float32
    Asrc = (jnp.eye(HEADS, dtype=f32)[:, None, :] * att_src[:, :, None]).reshape(D_MID, HEADS)
    Adst = (jnp.eye(HEADS, dtype=f32)[:, None, :] * att_dst[:, :, None]).reshape(D_MID, HEADS)
    A8 = jnp.concatenate([Asrc, Adst], axis=1)
    R = jnp.repeat(jnp.eye(HEADS, dtype=f32), D_HEAD, axis=1)

    ei1 = edge_index
    ei2 = sec_order_edge_index

    h, al8 = _tc_a(x, W1, A8)
    numer_c, small_c = _sc1_make()(h, al8, ei1, ei2)
    h2p = _tc_b(h, al8, numer_c, small_c, R, b1.reshape(1, D_MID), W2)
    out2_p = _sc2_make()(h2p, ei2)
    return _tc_c(out2_p, h2p, small_c, b2.reshape(1, D_OUT))
